# Initial kernel scaffold; baseline (speedup 1.0000x reference)
import jax
import jax.numpy as jnp
from jax.experimental import pallas as pl


def kernel(x, edge_source, edge_target, edge_attr, Wf, bf, Ws, bs, gamma, beta):
    raise NotImplementedError("write your pallas kernel here")


# speedup vs baseline: 2.4569x; 2.4569x over previous
import jax
import jax.numpy as jnp
from jax import lax
from jax.experimental import pallas as pl
from jax.experimental.pallas import tpu as pltpu
from jax.experimental.pallas import tpu_sc as plsc

N = 50000
E = 800000
D = 64
DE = 16
DP = 128

NC = 2
NS = 16
NW = NC * NS


NB = 1000


def _proj_body(x_ref, w1_ref, w2_ref, b_ref, t1_ref, t2_ref):
    xb = x_ref[...]
    t1_ref[...] = jnp.dot(xb, w1_ref[...], preferred_element_type=jnp.float32)
    t2_ref[...] = (
        jnp.dot(xb, w2_ref[...], preferred_element_type=jnp.float32) + b_ref[...]
    )


def _proj_tc(x, w1, w2, b):
    return pl.pallas_call(
        _proj_body,
        grid=(N // NB,),
        in_specs=[
            pl.BlockSpec((NB, D), lambda i: (i, 0)),
            pl.BlockSpec((D, DP), lambda i: (0, 0)),
            pl.BlockSpec((D, DP), lambda i: (0, 0)),
            pl.BlockSpec((1, DP), lambda i: (0, 0)),
        ],
        out_specs=[
            pl.BlockSpec((NB, DP), lambda i: (i, 0)),
            pl.BlockSpec((NB, DP), lambda i: (i, 0)),
        ],
        out_shape=[
            jax.ShapeDtypeStruct((N, DP), jnp.float32),
            jax.ShapeDtypeStruct((N, DP), jnp.float32),
        ],
    )(x, w1, w2, b)


EPW = E // NW
GC = 128
G_FULL = EPW // GC
G_REM = EPW - G_FULL * GC


def _gather_body(t1_hbm, t2_hbm, src_hbm, dst_hbm, a_hbm, b_hbm,
                 idx_s, idx_d, rows_s, rows_d,
                 idx_s2, idx_d2, rows_s2, rows_d2, sem):
    c = lax.axis_index("c")
    s = lax.axis_index("s")
    base = (c * NS + s) * EPW

    def do_chunk(off, n, isb, idb, rsb, rdb):
        pltpu.sync_copy(src_hbm.at[pl.ds(off, n)], isb)
        pltpu.sync_copy(dst_hbm.at[pl.ds(off, n)], idb)
        cp1 = pltpu.async_copy(t1_hbm.at[isb], rsb, sem)
        cp2 = pltpu.async_copy(t2_hbm.at[idb], rdb, sem)
        cp1.wait()
        cp2.wait()
        pltpu.sync_copy(rsb, a_hbm.at[pl.ds(off, n)])
        pltpu.sync_copy(rdb, b_hbm.at[pl.ds(off, n)])

    def loop_body(k, _):
        do_chunk(base + k * GC, GC, idx_s, idx_d, rows_s, rows_d)
        return ()

    lax.fori_loop(0, G_FULL, loop_body, ())
    do_chunk(base + G_FULL * GC, G_REM, idx_s2, idx_d2, rows_s2, rows_d2)


def _gather_sc(t1, t2, src, dst):
    mesh = plsc.VectorSubcoreMesh(core_axis_name="c", subcore_axis_name="s")
    f = pl.kernel(
        _gather_body,
        out_type=(jax.ShapeDtypeStruct((E, DP), jnp.float32),
                  jax.ShapeDtypeStruct((E, DP), jnp.float32)),
        mesh=mesh,
        scratch_types=[
            pltpu.VMEM((GC,), jnp.int32),
            pltpu.VMEM((GC,), jnp.int32),
            pltpu.VMEM((GC, DP), jnp.float32),
            pltpu.VMEM((GC, DP), jnp.float32),
            pltpu.VMEM((G_REM,), jnp.int32),
            pltpu.VMEM((G_REM,), jnp.int32),
            pltpu.VMEM((G_REM, DP), jnp.float32),
            pltpu.VMEM((G_REM, DP), jnp.float32),
            pltpu.SemaphoreType.DMA,
        ],
    )
    return f(t1, t2, src, dst)


EB = 4000


def _edge_body(a_ref, b_ref, ea_ref, a3_ref, m_ref):
    logits = (
        a_ref[...] + b_ref[...]
        + jnp.dot(ea_ref[...], a3_ref[...], preferred_element_type=jnp.float32)
    )
    f = jax.nn.sigmoid(logits[:, :D])
    s = jax.nn.softplus(logits[:, D:])
    m_ref[...] = f * s


def _edge_tc(a, b, ea, a3):
    return pl.pallas_call(
        _edge_body,
        grid=(E // EB,),
        in_specs=[
            pl.BlockSpec((EB, DP), lambda i: (i, 0)),
            pl.BlockSpec((EB, DP), lambda i: (i, 0)),
            pl.BlockSpec((EB, DE), lambda i: (i, 0)),
            pl.BlockSpec((DE, DP), lambda i: (0, 0)),
        ],
        out_specs=pl.BlockSpec((EB, D), lambda i: (i, 0)),
        out_shape=jax.ShapeDtypeStruct((E, D), jnp.float32),
    )(a, b, ea, a3)


NPC = N // NC
ACC_ROWS = 26624
ZPT = ACC_ROWS // NS
EPT = E // NS
S_FULL = EPT // GC
S_REM = EPT - S_FULL * GC
OC = 200
NOC = NPC // OC


def _scatter_body(m_hbm, src_hbm, msg_hbm, acc,
                  srcbuf, mbuf, idxbuf, srcbuf2, mbuf2, idxbuf2, zbuf):
    c = lax.axis_index("c")
    s = lax.axis_index("s")
    nodebase = c * NPC

    def zrow(r, _):
        for j in range(D // 16):
            zbuf[r, pl.ds(j * 16, 16)] = jnp.zeros((16,), jnp.float32)
        return ()
    lax.fori_loop(0, GC, zrow, ())
    for j in range(ZPT // GC):
        pltpu.sync_copy(zbuf, acc.at[pl.ds(s * ZPT + j * GC, GC)])
    plsc.subcore_barrier()

    def do_chunk(off, n, sb, mb, ib):
        pltpu.sync_copy(src_hbm.at[pl.ds(off, n)], sb)
        pltpu.sync_copy(m_hbm.at[pl.ds(off, n)], mb)
        for j in range(n // 16):
            v = sb[pl.ds(j * 16, 16)] - nodebase
            ok = (v >= 0) & (v < NPC)
            ib[pl.ds(j * 16, 16)] = jnp.where(ok, v, NPC)
        pltpu.sync_copy(mb, acc.at[ib], add=True)

    def loop_body(k, _):
        do_chunk(s * EPT + k * GC, GC, srcbuf, mbuf, idxbuf)
        return ()
    lax.fori_loop(0, S_FULL, loop_body, ())
    do_chunk(s * EPT + S_FULL * GC, S_REM, srcbuf2, mbuf2, idxbuf2)

    plsc.subcore_barrier()

    for i in range(8):
        cid = s * 8 + i

        @pl.when(cid < NOC)
        def _():
            pltpu.sync_copy(acc.at[pl.ds(cid * OC, OC)],
                            msg_hbm.at[pl.ds(nodebase + cid * OC, OC)])


def _scatter_sc(m, src):
    mesh = plsc.VectorSubcoreMesh(core_axis_name="c", subcore_axis_name="s")
    f = pl.kernel(
        _scatter_body,
        out_type=jax.ShapeDtypeStruct((N, D), jnp.float32),
        mesh=mesh,
        compiler_params=pltpu.CompilerParams(use_tc_tiling_on_sc=False),
        scratch_types=[
            pltpu.VMEM_SHARED((ACC_ROWS, D), jnp.float32),
            pltpu.VMEM((GC,), jnp.int32),
            pltpu.VMEM((GC, D), jnp.float32),
            pltpu.VMEM((GC,), jnp.int32),
            pltpu.VMEM((S_REM,), jnp.int32),
            pltpu.VMEM((S_REM, D), jnp.float32),
            pltpu.VMEM((S_REM,), jnp.int32),
            pltpu.VMEM((GC, D), jnp.float32),
        ],
    )
    return f(m, src)


def _stats_body(msg_ref, out_ref):
    @pl.when(pl.program_id(0) == 0)
    def _():
        out_ref[...] = jnp.zeros_like(out_ref)

    blk = msg_ref[...]
    s1 = jnp.sum(blk, axis=0, keepdims=True)
    s2 = jnp.sum(blk * blk, axis=0, keepdims=True)
    out_ref[...] += jnp.concatenate([s1, s2], axis=0)


def _stats_tc(msg):
    return pl.pallas_call(
        _stats_body,
        grid=(N // NB,),
        in_specs=[pl.BlockSpec((NB, D), lambda i: (i, 0))],
        out_specs=pl.BlockSpec((2, D), lambda i: (0, 0)),
        out_shape=jax.ShapeDtypeStruct((2, D), jnp.float32),
    )(msg)


def _final_body(x_ref, msg_ref, sums_ref, g_ref, bt_ref, out_ref):
    mean = sums_ref[0:1, :] * (1.0 / N)
    ex2 = sums_ref[1:2, :] * (1.0 / N)
    var = ex2 - mean * mean
    inv = lax.rsqrt(var + 1e-5)
    normed = (msg_ref[...] - mean) * (inv * g_ref[...]) + bt_ref[...]
    out_ref[...] = jax.nn.softplus(x_ref[...] + normed)


def _final_tc(x, msg, sums, g, bt):
    return pl.pallas_call(
        _final_body,
        grid=(N // NB,),
        in_specs=[
            pl.BlockSpec((NB, D), lambda i: (i, 0)),
            pl.BlockSpec((NB, D), lambda i: (i, 0)),
            pl.BlockSpec((2, D), lambda i: (0, 0)),
            pl.BlockSpec((1, D), lambda i: (0, 0)),
            pl.BlockSpec((1, D), lambda i: (0, 0)),
        ],
        out_specs=pl.BlockSpec((NB, D), lambda i: (i, 0)),
        out_shape=jax.ShapeDtypeStruct((N, D), jnp.float32),
    )(x, msg, sums, g, bt)


def kernel(x, edge_source, edge_target, edge_attr, Wf, bf, Ws, bs, gamma, beta):
    src = edge_source.astype(jnp.int32)
    dst = edge_target.astype(jnp.int32)
    w1 = jnp.concatenate([Wf[:, :D].T, Ws[:, :D].T], axis=1)
    w2 = jnp.concatenate([Wf[:, D:2 * D].T, Ws[:, D:2 * D].T], axis=1)
    a3 = jnp.concatenate([Wf[:, 2 * D:].T, Ws[:, 2 * D:].T], axis=1)
    b = jnp.concatenate([bf, bs]).reshape(1, DP)

    t1, t2 = _proj_tc(x, w1, w2, b)
    a, bb = _gather_sc(t1, t2, src, dst)
    m = _edge_tc(a, bb, edge_attr, a3)
    msg = _scatter_sc(m, src)
    sums = _stats_tc(msg)
    return _final_tc(x, msg, sums, gamma.reshape(1, D), beta.reshape(1, D))


# speedup vs baseline: 2.9204x; 1.1886x over previous
import jax
import jax.numpy as jnp
from jax import lax
from jax.experimental import pallas as pl
from jax.experimental.pallas import tpu as pltpu
from jax.experimental.pallas import tpu_sc as plsc

N = 50000
E = 800000
D = 64
DE = 16
DP = 128

NC = 2
NS = 16
NW = NC * NS


NB = 1000


def _proj_body(x_ref, w1_ref, w2_ref, b_ref, t1_ref, t2_ref):
    xb = x_ref[...]
    t1_ref[...] = jnp.dot(xb, w1_ref[...], preferred_element_type=jnp.float32)
    t2_ref[...] = (
        jnp.dot(xb, w2_ref[...], preferred_element_type=jnp.float32) + b_ref[...]
    )


def _proj_tc(x, w1, w2, b):
    return pl.pallas_call(
        _proj_body,
        grid=(N // NB,),
        in_specs=[
            pl.BlockSpec((NB, D), lambda i: (i, 0)),
            pl.BlockSpec((D, DP), lambda i: (0, 0)),
            pl.BlockSpec((D, DP), lambda i: (0, 0)),
            pl.BlockSpec((1, DP), lambda i: (0, 0)),
        ],
        out_specs=[
            pl.BlockSpec((NB, DP), lambda i: (i, 0)),
            pl.BlockSpec((NB, DP), lambda i: (i, 0)),
        ],
        out_shape=[
            jax.ShapeDtypeStruct((N, DP), jnp.float32),
            jax.ShapeDtypeStruct((N, DP), jnp.float32),
        ],
    )(x, w1, w2, b)


GC = 128
NCHUNK = E // GC
RING = 3
G_FULL = (NCHUNK // NW // RING) * RING
G_TAIL = NCHUNK - G_FULL * NW


def _gather_body(t1_hbm, t2_hbm, src_hbm, dst_hbm, a_hbm, b_hbm,
                 idx_s, idx_d, rows_s, rows_d,
                 sem_is, sem_id, sem_g, sem_ws, sem_wd):
    c = lax.axis_index("c")
    s = lax.axis_index("s")
    w = c * NS + s
    base = w * G_FULL

    def idx_load(k, b):
        off = (base + k) * GC
        pltpu.async_copy(src_hbm.at[pl.ds(off, GC)], idx_s.at[b], sem_is)
        pltpu.async_copy(dst_hbm.at[pl.ds(off, GC)], idx_d.at[b], sem_id)

    for b in range(RING):
        idx_load(b, b)

    def group(g, _):
        for b in range(RING):
            pltpu.make_async_copy(src_hbm.at[pl.ds(0, GC)], idx_s.at[b],
                                  sem_is).wait()
            pltpu.make_async_copy(dst_hbm.at[pl.ds(0, GC)], idx_d.at[b],
                                  sem_id).wait()
            if_first = g == 0

            @pl.when(jnp.logical_not(if_first))
            def _():
                pltpu.make_async_copy(rows_s.at[b],
                                      a_hbm.at[pl.ds(0, GC)], sem_ws).wait()
                pltpu.make_async_copy(rows_d.at[b],
                                      b_hbm.at[pl.ds(0, GC)], sem_wd).wait()
            pltpu.async_copy(t1_hbm.at[idx_s.at[b]], rows_s.at[b], sem_g)
            pltpu.async_copy(t2_hbm.at[idx_d.at[b]], rows_d.at[b], sem_g)
        for b in range(RING):
            k = g * RING + b
            off = (base + k) * GC
            pltpu.make_async_copy(t1_hbm.at[idx_s.at[b]], rows_s.at[b],
                                  sem_g).wait()
            pltpu.make_async_copy(t2_hbm.at[idx_d.at[b]], rows_d.at[b],
                                  sem_g).wait()
            pltpu.async_copy(rows_s.at[b], a_hbm.at[pl.ds(off, GC)], sem_ws)
            pltpu.async_copy(rows_d.at[b], b_hbm.at[pl.ds(off, GC)], sem_wd)

            @pl.when(k + RING < G_FULL)
            def _():
                idx_load(k + RING, b)
        return ()

    lax.fori_loop(0, G_FULL // RING, group, ())
    for b in range(RING):
        pltpu.make_async_copy(rows_s.at[b], a_hbm.at[pl.ds(0, GC)],
                              sem_ws).wait()
        pltpu.make_async_copy(rows_d.at[b], b_hbm.at[pl.ds(0, GC)],
                              sem_wd).wait()

    @pl.when(w < G_TAIL)
    def _():
        off = (G_FULL * NW + w) * GC
        pltpu.sync_copy(src_hbm.at[pl.ds(off, GC)], idx_s.at[0])
        pltpu.sync_copy(dst_hbm.at[pl.ds(off, GC)], idx_d.at[0])
        cp1 = pltpu.async_copy(t1_hbm.at[idx_s.at[0]], rows_s.at[0], sem_g)
        cp2 = pltpu.async_copy(t2_hbm.at[idx_d.at[0]], rows_d.at[0], sem_g)
        cp1.wait()
        cp2.wait()
        pltpu.sync_copy(rows_s.at[0], a_hbm.at[pl.ds(off, GC)])
        pltpu.sync_copy(rows_d.at[0], b_hbm.at[pl.ds(off, GC)])


def _gather_sc(t1, t2, src, dst):
    mesh = plsc.VectorSubcoreMesh(core_axis_name="c", subcore_axis_name="s")
    f = pl.kernel(
        _gather_body,
        out_type=(jax.ShapeDtypeStruct((E, DP), jnp.float32),
                  jax.ShapeDtypeStruct((E, DP), jnp.float32)),
        mesh=mesh,
        scratch_types=[
            pltpu.VMEM((RING, GC), jnp.int32),
            pltpu.VMEM((RING, GC), jnp.int32),
            pltpu.VMEM((RING, GC, DP), jnp.float32),
            pltpu.VMEM((RING, GC, DP), jnp.float32),
            pltpu.SemaphoreType.DMA,
            pltpu.SemaphoreType.DMA,
            pltpu.SemaphoreType.DMA,
            pltpu.SemaphoreType.DMA,
            pltpu.SemaphoreType.DMA,
        ],
    )
    return f(t1, t2, src, dst)


EB = 4000


def _edge_body(a_ref, b_ref, ea_ref, a3_ref, m_ref):
    logits = (
        a_ref[...] + b_ref[...]
        + jnp.dot(ea_ref[...], a3_ref[...], preferred_element_type=jnp.float32)
    )
    f = jax.nn.sigmoid(logits[:, :D])
    s = jax.nn.softplus(logits[:, D:])
    m_ref[...] = f * s


def _edge_tc(a, b, ea, a3):
    return pl.pallas_call(
        _edge_body,
        grid=(E // EB,),
        in_specs=[
            pl.BlockSpec((EB, DP), lambda i: (i, 0)),
            pl.BlockSpec((EB, DP), lambda i: (i, 0)),
            pl.BlockSpec((EB, DE), lambda i: (i, 0)),
            pl.BlockSpec((DE, DP), lambda i: (0, 0)),
        ],
        out_specs=pl.BlockSpec((EB, D), lambda i: (i, 0)),
        out_shape=jax.ShapeDtypeStruct((E, D), jnp.float32),
    )(a, b, ea, a3)


NPC = N // NC
ACC_ROWS = 25088
ZPT = ACC_ROWS // NS
ZC = 56
S_FULL = (NCHUNK // NS // RING) * RING
S_TAIL = NCHUNK - S_FULL * NS
OC = 200
NOC = NPC // OC


def _scatter_body(m_hbm, src_hbm, msg_hbm, acc,
                  srcbuf, mbuf, idxbuf, zbuf, sem_s, sem_m, sem_sc):
    c = lax.axis_index("c")
    s = lax.axis_index("s")
    nodebase = c * NPC
    base = s * S_FULL

    def zrow(r, _):
        for j in range(D // 16):
            zbuf[r, pl.ds(j * 16, 16)] = jnp.zeros((16,), jnp.float32)
        return ()
    lax.fori_loop(0, ZC, zrow, ())
    for j in range(ZPT // ZC):
        pltpu.sync_copy(zbuf, acc.at[pl.ds(s * ZPT + j * ZC, ZC)])
    plsc.subcore_barrier()

    def loads(k, b):
        off = (base + k) * GC
        pltpu.async_copy(src_hbm.at[pl.ds(off, GC)], srcbuf.at[b], sem_s)
        pltpu.async_copy(m_hbm.at[pl.ds(off, GC)], mbuf.at[b], sem_m)

    def remap(b):
        sb = srcbuf.at[b]
        ib = idxbuf.at[b]
        for j in range(GC // 16):
            v = sb[pl.ds(j * 16, 16)] - nodebase
            ok = (v >= 0) & (v < NPC)
            ib[pl.ds(j * 16, 16)] = jnp.where(ok, v, NPC)

    for b in range(RING):
        loads(b, b)

    def group(g, _):
        cps = []
        for b in range(RING):
            pltpu.make_async_copy(src_hbm.at[pl.ds(0, GC)], srcbuf.at[b],
                                  sem_s).wait()
            remap(b)
            pltpu.make_async_copy(m_hbm.at[pl.ds(0, GC)], mbuf.at[b],
                                  sem_m).wait()
            cps.append(pltpu.async_copy(mbuf.at[b], acc.at[idxbuf.at[b]],
                                        sem_sc, add=True))
        for b in range(RING):
            k = g * RING + b
            cps[b].wait()

            @pl.when(k + RING < S_FULL)
            def _():
                loads(k + RING, b)
        return ()

    lax.fori_loop(0, S_FULL // RING, group, ())

    @pl.when(s < S_TAIL)
    def _():
        off = (S_FULL * NS + s) * GC
        pltpu.sync_copy(src_hbm.at[pl.ds(off, GC)], srcbuf.at[0])
        pltpu.sync_copy(m_hbm.at[pl.ds(off, GC)], mbuf.at[0])
        remap(0)
        pltpu.sync_copy(mbuf.at[0], acc.at[idxbuf.at[0]], add=True)

    plsc.subcore_barrier()

    for i in range(8):
        cid = s * 8 + i

        @pl.when(cid < NOC)
        def _():
            pltpu.sync_copy(acc.at[pl.ds(cid * OC, OC)],
                            msg_hbm.at[pl.ds(nodebase + cid * OC, OC)])


def _scatter_sc(m, src):
    mesh = plsc.VectorSubcoreMesh(core_axis_name="c", subcore_axis_name="s")
    f = pl.kernel(
        _scatter_body,
        out_type=jax.ShapeDtypeStruct((N, D), jnp.float32),
        mesh=mesh,
        compiler_params=pltpu.CompilerParams(use_tc_tiling_on_sc=False),
        scratch_types=[
            pltpu.VMEM_SHARED((ACC_ROWS, D), jnp.float32),
            pltpu.VMEM((RING, GC), jnp.int32),
            pltpu.VMEM((RING, GC, D), jnp.float32),
            pltpu.VMEM((RING, GC), jnp.int32),
            pltpu.VMEM((ZC, D), jnp.float32),
            pltpu.SemaphoreType.DMA,
            pltpu.SemaphoreType.DMA,
            pltpu.SemaphoreType.DMA,
        ],
    )
    return f(m, src)


def _stats_body(msg_ref, out_ref):
    @pl.when(pl.program_id(0) == 0)
    def _():
        out_ref[...] = jnp.zeros_like(out_ref)

    blk = msg_ref[...]
    s1 = jnp.sum(blk, axis=0, keepdims=True)
    s2 = jnp.sum(blk * blk, axis=0, keepdims=True)
    out_ref[...] += jnp.concatenate([s1, s2], axis=0)


def _stats_tc(msg):
    return pl.pallas_call(
        _stats_body,
        grid=(N // NB,),
        in_specs=[pl.BlockSpec((NB, D), lambda i: (i, 0))],
        out_specs=pl.BlockSpec((2, D), lambda i: (0, 0)),
        out_shape=jax.ShapeDtypeStruct((2, D), jnp.float32),
    )(msg)


def _final_body(x_ref, msg_ref, sums_ref, g_ref, bt_ref, out_ref):
    mean = sums_ref[0:1, :] * (1.0 / N)
    ex2 = sums_ref[1:2, :] * (1.0 / N)
    var = ex2 - mean * mean
    inv = lax.rsqrt(var + 1e-5)
    normed = (msg_ref[...] - mean) * (inv * g_ref[...]) + bt_ref[...]
    out_ref[...] = jax.nn.softplus(x_ref[...] + normed)


def _final_tc(x, msg, sums, g, bt):
    return pl.pallas_call(
        _final_body,
        grid=(N // NB,),
        in_specs=[
            pl.BlockSpec((NB, D), lambda i: (i, 0)),
            pl.BlockSpec((NB, D), lambda i: (i, 0)),
            pl.BlockSpec((2, D), lambda i: (0, 0)),
            pl.BlockSpec((1, D), lambda i: (0, 0)),
            pl.BlockSpec((1, D), lambda i: (0, 0)),
        ],
        out_specs=pl.BlockSpec((NB, D), lambda i: (i, 0)),
        out_shape=jax.ShapeDtypeStruct((N, D), jnp.float32),
    )(x, msg, sums, g, bt)


def kernel(x, edge_source, edge_target, edge_attr, Wf, bf, Ws, bs, gamma, beta):
    src = edge_source.astype(jnp.int32)
    dst = edge_target.astype(jnp.int32)
    w1 = jnp.concatenate([Wf[:, :D].T, Ws[:, :D].T], axis=1)
    w2 = jnp.concatenate([Wf[:, D:2 * D].T, Ws[:, D:2 * D].T], axis=1)
    a3 = jnp.concatenate([Wf[:, 2 * D:].T, Ws[:, 2 * D:].T], axis=1)
    b = jnp.concatenate([bf, bs]).reshape(1, DP)

    t1, t2 = _proj_tc(x, w1, w2, b)
    a, bb = _gather_sc(t1, t2, src, dst)
    m = _edge_tc(a, bb, edge_attr, a3)
    msg = _scatter_sc(m, src)
    sums = _stats_tc(msg)
    return _final_tc(x, msg, sums, gamma.reshape(1, D), beta.reshape(1, D))


# speedup vs baseline: 3.1886x; 1.0918x over previous
import functools

import jax
import jax.numpy as jnp
from jax import lax
from jax.experimental import pallas as pl
from jax.experimental.pallas import tpu as pltpu
from jax.experimental.pallas import tpu_sc as plsc

N = 50000
E = 800000
D = 64
DE = 16
DP = 128

NC = 2
NS = 16
NW = NC * NS

NHALF = 2
E2 = E // NHALF


NB = 1000


def _proj_body(x_ref, w1_ref, w2_ref, b_ref, t1_ref, t2_ref):
    xb = x_ref[...]
    t1_ref[...] = jnp.dot(xb, w1_ref[...], preferred_element_type=jnp.float32)
    t2_ref[...] = (
        jnp.dot(xb, w2_ref[...], preferred_element_type=jnp.float32) + b_ref[...]
    )


def _proj_tc(x, w1, w2, b):
    return pl.pallas_call(
        _proj_body,
        grid=(N // NB,),
        in_specs=[
            pl.BlockSpec((NB, D), lambda i: (i, 0)),
            pl.BlockSpec((D, DP), lambda i: (0, 0)),
            pl.BlockSpec((D, DP), lambda i: (0, 0)),
            pl.BlockSpec((1, DP), lambda i: (0, 0)),
        ],
        out_specs=[
            pl.BlockSpec((NB, DP), lambda i: (i, 0)),
            pl.BlockSpec((NB, DP), lambda i: (i, 0)),
        ],
        out_shape=[
            jax.ShapeDtypeStruct((N, DP), jnp.float32),
            jax.ShapeDtypeStruct((N, DP), jnp.float32),
        ],
    )(x, w1, w2, b)


GC = 128
RING = 3
NCH = E2 // GC
G_FULL = (NCH // (NW * RING)) * RING
G_TAIL = NCH - G_FULL * NW
G_TR = -(-G_TAIL // NW)


def _gather_body(chunk0, t1_hbm, t2_hbm, src_hbm, dst_hbm, a_hbm, b_hbm,
                 idx_s, idx_d, rows_s, rows_d,
                 sem_is, sem_id, sem_g, sem_ws, sem_wd):
    c = lax.axis_index("c")
    s = lax.axis_index("s")
    w = c * NS + s
    wbase = w * G_FULL

    def idx_load(k, b):
        off = (chunk0 + wbase + k) * GC
        pltpu.async_copy(src_hbm.at[pl.ds(off, GC)], idx_s.at[b], sem_is)
        pltpu.async_copy(dst_hbm.at[pl.ds(off, GC)], idx_d.at[b], sem_id)

    for b in range(RING):
        idx_load(b, b)

    def group(g, _):
        for b in range(RING):
            pltpu.make_async_copy(src_hbm.at[pl.ds(0, GC)], idx_s.at[b],
                                  sem_is).wait()
            pltpu.make_async_copy(dst_hbm.at[pl.ds(0, GC)], idx_d.at[b],
                                  sem_id).wait()

            @pl.when(g != 0)
            def _():
                pltpu.make_async_copy(rows_s.at[b],
                                      a_hbm.at[pl.ds(0, GC)], sem_ws).wait()
                pltpu.make_async_copy(rows_d.at[b],
                                      b_hbm.at[pl.ds(0, GC)], sem_wd).wait()
            pltpu.async_copy(t1_hbm.at[idx_s.at[b]], rows_s.at[b], sem_g)
            pltpu.async_copy(t2_hbm.at[idx_d.at[b]], rows_d.at[b], sem_g)
        for b in range(RING):
            k = g * RING + b
            off = (wbase + k) * GC
            pltpu.make_async_copy(t1_hbm.at[idx_s.at[b]], rows_s.at[b],
                                  sem_g).wait()
            pltpu.make_async_copy(t2_hbm.at[idx_d.at[b]], rows_d.at[b],
                                  sem_g).wait()
            pltpu.async_copy(rows_s.at[b], a_hbm.at[pl.ds(off, GC)], sem_ws)
            pltpu.async_copy(rows_d.at[b], b_hbm.at[pl.ds(off, GC)], sem_wd)

            @pl.when(k + RING < G_FULL)
            def _():
                idx_load(k + RING, b)
        return ()

    lax.fori_loop(0, G_FULL // RING, group, ())
    for b in range(RING):
        pltpu.make_async_copy(rows_s.at[b], a_hbm.at[pl.ds(0, GC)],
                              sem_ws).wait()
        pltpu.make_async_copy(rows_d.at[b], b_hbm.at[pl.ds(0, GC)],
                              sem_wd).wait()

    for t in range(G_TR):
        tid = t * NW + w

        @pl.when(tid < G_TAIL)
        def _():
            lk = G_FULL * NW + tid
            off = (chunk0 + lk) * GC
            loff = lk * GC
            pltpu.sync_copy(src_hbm.at[pl.ds(off, GC)], idx_s.at[0])
            pltpu.sync_copy(dst_hbm.at[pl.ds(off, GC)], idx_d.at[0])
            cp1 = pltpu.async_copy(t1_hbm.at[idx_s.at[0]], rows_s.at[0], sem_g)
            cp2 = pltpu.async_copy(t2_hbm.at[idx_d.at[0]], rows_d.at[0], sem_g)
            cp1.wait()
            cp2.wait()
            pltpu.sync_copy(rows_s.at[0], a_hbm.at[pl.ds(loff, GC)])
            pltpu.sync_copy(rows_d.at[0], b_hbm.at[pl.ds(loff, GC)])


def _gather_sc(t1, t2, src, dst, half):
    mesh = plsc.VectorSubcoreMesh(core_axis_name="c", subcore_axis_name="s")
    f = pl.kernel(
        functools.partial(_gather_body, half * NCH),
        out_type=(jax.ShapeDtypeStruct((E2, DP), jnp.float32),
                  jax.ShapeDtypeStruct((E2, DP), jnp.float32)),
        mesh=mesh,
        scratch_types=[
            pltpu.VMEM((RING, GC), jnp.int32),
            pltpu.VMEM((RING, GC), jnp.int32),
            pltpu.VMEM((RING, GC, DP), jnp.float32),
            pltpu.VMEM((RING, GC, DP), jnp.float32),
            pltpu.SemaphoreType.DMA,
            pltpu.SemaphoreType.DMA,
            pltpu.SemaphoreType.DMA,
            pltpu.SemaphoreType.DMA,
            pltpu.SemaphoreType.DMA,
        ],
    )
    return f(t1, t2, src, dst)


EB = 4000


def _edge_body(a_ref, b_ref, ea_ref, a3_ref, m_ref):
    logits = (
        a_ref[...] + b_ref[...]
        + jnp.dot(ea_ref[...], a3_ref[...], preferred_element_type=jnp.float32)
    )
    f = jax.nn.sigmoid(logits[:, :D])
    s = jax.nn.softplus(logits[:, D:])
    m_ref[...] = f * s


def _edge_tc(a, b, ea, a3, half):
    off = half * (E2 // EB)
    return pl.pallas_call(
        _edge_body,
        grid=(E2 // EB,),
        in_specs=[
            pl.BlockSpec((EB, DP), lambda i: (i, 0)),
            pl.BlockSpec((EB, DP), lambda i: (i, 0)),
            pl.BlockSpec((EB, DE), lambda i: (i + off, 0)),
            pl.BlockSpec((DE, DP), lambda i: (0, 0)),
        ],
        out_specs=pl.BlockSpec((EB, D), lambda i: (i, 0)),
        out_shape=jax.ShapeDtypeStruct((E2, D), jnp.float32),
    )(a, b, ea, a3)


NPC = N // NC
ACC_ROWS = 25088
ZPT = ACC_ROWS // NS
ZC = 56
S_FULL = (NCH // (NS * RING)) * RING
S_TAIL = NCH - S_FULL * NS
OC = 200
NOC = NPC // OC


def _scatter_body(chunk0, m_hbm, src_hbm, msg_hbm, acc,
                  srcbuf, mbuf, idxbuf, zbuf, sem_s, sem_m, sem_sc):
    c = lax.axis_index("c")
    s = lax.axis_index("s")
    nodebase = c * NPC
    sbase = s * S_FULL

    def zrow(r, _):
        for j in range(D // 16):
            zbuf[r, pl.ds(j * 16, 16)] = jnp.zeros((16,), jnp.float32)
        return ()
    lax.fori_loop(0, ZC, zrow, ())
    for j in range(ZPT // ZC):
        pltpu.sync_copy(zbuf, acc.at[pl.ds(s * ZPT + j * ZC, ZC)])
    plsc.subcore_barrier()

    def loads(k, b):
        goff = (chunk0 + sbase + k) * GC
        loff = (sbase + k) * GC
        pltpu.async_copy(src_hbm.at[pl.ds(goff, GC)], srcbuf.at[b], sem_s)
        pltpu.async_copy(m_hbm.at[pl.ds(loff, GC)], mbuf.at[b], sem_m)

    def remap(b):
        sb = srcbuf.at[b]
        ib = idxbuf.at[b]
        for j in range(GC // 16):
            v = sb[pl.ds(j * 16, 16)] - nodebase
            ok = (v >= 0) & (v < NPC)
            ib[pl.ds(j * 16, 16)] = jnp.where(ok, v, NPC)

    for b in range(RING):
        loads(b, b)

    def group(g, _):
        cps = []
        for b in range(RING):
            pltpu.make_async_copy(src_hbm.at[pl.ds(0, GC)], srcbuf.at[b],
                                  sem_s).wait()
            remap(b)
            pltpu.make_async_copy(m_hbm.at[pl.ds(0, GC)], mbuf.at[b],
                                  sem_m).wait()
            cps.append(pltpu.async_copy(mbuf.at[b], acc.at[idxbuf.at[b]],
                                        sem_sc, add=True))
        for b in range(RING):
            k = g * RING + b
            cps[b].wait()

            @pl.when(k + RING < S_FULL)
            def _():
                loads(k + RING, b)
        return ()

    lax.fori_loop(0, S_FULL // RING, group, ())

    @pl.when(s < S_TAIL)
    def _():
        lk = S_FULL * NS + s
        goff = (chunk0 + lk) * GC
        loff = lk * GC
        pltpu.sync_copy(src_hbm.at[pl.ds(goff, GC)], srcbuf.at[0])
        pltpu.sync_copy(m_hbm.at[pl.ds(loff, GC)], mbuf.at[0])
        remap(0)
        pltpu.sync_copy(mbuf.at[0], acc.at[idxbuf.at[0]], add=True)

    plsc.subcore_barrier()

    for i in range(8):
        cid = s * 8 + i

        @pl.when(cid < NOC)
        def _():
            pltpu.sync_copy(acc.at[pl.ds(cid * OC, OC)],
                            msg_hbm.at[pl.ds(nodebase + cid * OC, OC)])


def _scatter_sc(m, src, half):
    mesh = plsc.VectorSubcoreMesh(core_axis_name="c", subcore_axis_name="s")
    f = pl.kernel(
        functools.partial(_scatter_body, half * NCH),
        out_type=jax.ShapeDtypeStruct((N, D), jnp.float32),
        mesh=mesh,
        compiler_params=pltpu.CompilerParams(use_tc_tiling_on_sc=False),
        scratch_types=[
            pltpu.VMEM_SHARED((ACC_ROWS, D), jnp.float32),
            pltpu.VMEM((RING, GC), jnp.int32),
            pltpu.VMEM((RING, GC, D), jnp.float32),
            pltpu.VMEM((RING, GC), jnp.int32),
            pltpu.VMEM((ZC, D), jnp.float32),
            pltpu.SemaphoreType.DMA,
            pltpu.SemaphoreType.DMA,
            pltpu.SemaphoreType.DMA,
        ],
    )
    return f(m, src)


def _stats_body(ma_ref, mb_ref, out_ref):
    @pl.when(pl.program_id(0) == 0)
    def _():
        out_ref[...] = jnp.zeros_like(out_ref)

    blk = ma_ref[...] + mb_ref[...]
    s1 = jnp.sum(blk, axis=0, keepdims=True)
    s2 = jnp.sum(blk * blk, axis=0, keepdims=True)
    out_ref[...] += jnp.concatenate([s1, s2], axis=0)


def _stats_tc(msga, msgb):
    return pl.pallas_call(
        _stats_body,
        grid=(N // NB,),
        in_specs=[pl.BlockSpec((NB, D), lambda i: (i, 0)),
                  pl.BlockSpec((NB, D), lambda i: (i, 0))],
        out_specs=pl.BlockSpec((2, D), lambda i: (0, 0)),
        out_shape=jax.ShapeDtypeStruct((2, D), jnp.float32),
    )(msga, msgb)


def _final_body(x_ref, ma_ref, mb_ref, sums_ref, g_ref, bt_ref, out_ref):
    mean = sums_ref[0:1, :] * (1.0 / N)
    ex2 = sums_ref[1:2, :] * (1.0 / N)
    var = ex2 - mean * mean
    inv = lax.rsqrt(var + 1e-5)
    msg = ma_ref[...] + mb_ref[...]
    normed = (msg - mean) * (inv * g_ref[...]) + bt_ref[...]
    out_ref[...] = jax.nn.softplus(x_ref[...] + normed)


def _final_tc(x, msga, msgb, sums, g, bt):
    return pl.pallas_call(
        _final_body,
        grid=(N // NB,),
        in_specs=[
            pl.BlockSpec((NB, D), lambda i: (i, 0)),
            pl.BlockSpec((NB, D), lambda i: (i, 0)),
            pl.BlockSpec((NB, D), lambda i: (i, 0)),
            pl.BlockSpec((2, D), lambda i: (0, 0)),
            pl.BlockSpec((1, D), lambda i: (0, 0)),
            pl.BlockSpec((1, D), lambda i: (0, 0)),
        ],
        out_specs=pl.BlockSpec((NB, D), lambda i: (i, 0)),
        out_shape=jax.ShapeDtypeStruct((N, D), jnp.float32),
    )(x, msga, msgb, sums, g, bt)


def kernel(x, edge_source, edge_target, edge_attr, Wf, bf, Ws, bs, gamma, beta):
    src = edge_source.astype(jnp.int32)
    dst = edge_target.astype(jnp.int32)
    w1 = jnp.concatenate([Wf[:, :D].T, Ws[:, :D].T], axis=1)
    w2 = jnp.concatenate([Wf[:, D:2 * D].T, Ws[:, D:2 * D].T], axis=1)
    a3 = jnp.concatenate([Wf[:, 2 * D:].T, Ws[:, 2 * D:].T], axis=1)
    b = jnp.concatenate([bf, bs]).reshape(1, DP)

    t1, t2 = _proj_tc(x, w1, w2, b)
    a0, b0 = _gather_sc(t1, t2, src, dst, 0)
    a1, b1 = _gather_sc(t1, t2, src, dst, 1)
    m0 = _edge_tc(a0, b0, edge_attr, a3, 0)
    m1 = _edge_tc(a1, b1, edge_attr, a3, 1)
    msg0 = _scatter_sc(m0, src, 0)
    msg1 = _scatter_sc(m1, src, 1)
    sums = _stats_tc(msg0, msg1)
    return _final_tc(x, msg0, msg1, sums, gamma.reshape(1, D), beta.reshape(1, D))


# speedup vs baseline: 3.6731x; 1.1519x over previous
import functools

import jax
import jax.numpy as jnp
from jax import lax
from jax.experimental import pallas as pl
from jax.experimental.pallas import tpu as pltpu
from jax.experimental.pallas import tpu_sc as plsc

N = 50000
E = 800000
D = 64
DE = 16
DP = 128

NC = 2
NS = 16
NW = NC * NS

NHALF = 2
E2 = E // NHALF


NB = 1000


def _proj_body(x_ref, w1_ref, w2_ref, b_ref, t1_ref, t2_ref):
    xb = x_ref[...]
    t1_ref[...] = jnp.dot(xb, w1_ref[...], preferred_element_type=jnp.float32)
    t2_ref[...] = (
        jnp.dot(xb, w2_ref[...], preferred_element_type=jnp.float32) + b_ref[...]
    )


def _proj_tc(x, w1, w2, b):
    return pl.pallas_call(
        _proj_body,
        grid=(N // NB,),
        in_specs=[
            pl.BlockSpec((NB, D), lambda i: (i, 0)),
            pl.BlockSpec((D, DP), lambda i: (0, 0)),
            pl.BlockSpec((D, DP), lambda i: (0, 0)),
            pl.BlockSpec((1, DP), lambda i: (0, 0)),
        ],
        out_specs=[
            pl.BlockSpec((NB, DP), lambda i: (i, 0)),
            pl.BlockSpec((NB, DP), lambda i: (i, 0)),
        ],
        out_shape=[
            jax.ShapeDtypeStruct((N, DP), jnp.float32),
            jax.ShapeDtypeStruct((N, DP), jnp.float32),
        ],
    )(x, w1, w2, b)


GC = 128
RING = 3
NCH = E2 // GC
G_FULL = (NCH // (NW * RING)) * RING
G_TAIL = NCH - G_FULL * NW
G_TR = -(-G_TAIL // NW)


def _gather_body(chunk0, t1_hbm, t2_hbm, src_hbm, dst_hbm, ab_hbm,
                 idx_s, idx_d, rows,
                 sem_is, sem_id, sem_g1, sem_g2, sem_wb):
    c = lax.axis_index("c")
    s = lax.axis_index("s")
    w = c * NS + s
    wbase = w * G_FULL

    def idx_load(k, b):
        off = (chunk0 + wbase + k) * GC
        pltpu.async_copy(src_hbm.at[pl.ds(off, GC)], idx_s.at[b], sem_is)
        pltpu.async_copy(dst_hbm.at[pl.ds(off, GC)], idx_d.at[b], sem_id)

    for b in range(RING):
        idx_load(b, b)

    def group(g, _):
        for b in range(RING):
            pltpu.make_async_copy(src_hbm.at[pl.ds(0, GC)], idx_s.at[b],
                                  sem_is).wait()

            @pl.when(g != 0)
            def _():
                pltpu.make_async_copy(rows.at[b],
                                      ab_hbm.at[pl.ds(0, GC)], sem_wb).wait()
            pltpu.async_copy(t1_hbm.at[idx_s.at[b]], rows.at[b], sem_g1)
        for b in range(RING):
            pltpu.make_async_copy(t1_hbm.at[idx_s.at[b]], rows.at[b],
                                  sem_g1).wait()
            pltpu.make_async_copy(dst_hbm.at[pl.ds(0, GC)], idx_d.at[b],
                                  sem_id).wait()
            pltpu.async_copy(t2_hbm.at[idx_d.at[b]], rows.at[b], sem_g2,
                             add=True)
        for b in range(RING):
            k = g * RING + b
            off = (wbase + k) * GC
            pltpu.make_async_copy(t2_hbm.at[idx_d.at[b]], rows.at[b],
                                  sem_g2).wait()
            pltpu.async_copy(rows.at[b], ab_hbm.at[pl.ds(off, GC)], sem_wb)

            @pl.when(k + RING < G_FULL)
            def _():
                idx_load(k + RING, b)
        return ()

    lax.fori_loop(0, G_FULL // RING, group, ())
    for b in range(RING):
        pltpu.make_async_copy(rows.at[b], ab_hbm.at[pl.ds(0, GC)],
                              sem_wb).wait()

    for t in range(G_TR):
        tid = t * NW + w

        @pl.when(tid < G_TAIL)
        def _():
            lk = G_FULL * NW + tid
            off = (chunk0 + lk) * GC
            loff = lk * GC
            pltpu.sync_copy(src_hbm.at[pl.ds(off, GC)], idx_s.at[0])
            pltpu.sync_copy(dst_hbm.at[pl.ds(off, GC)], idx_d.at[0])
            pltpu.async_copy(t1_hbm.at[idx_s.at[0]], rows.at[0], sem_g1).wait()
            pltpu.async_copy(t2_hbm.at[idx_d.at[0]], rows.at[0], sem_g2,
                             add=True).wait()
            pltpu.sync_copy(rows.at[0], ab_hbm.at[pl.ds(loff, GC)])


def _gather_sc(t1, t2, src, dst, half):
    mesh = plsc.VectorSubcoreMesh(core_axis_name="c", subcore_axis_name="s")
    f = pl.kernel(
        functools.partial(_gather_body, half * NCH),
        out_type=jax.ShapeDtypeStruct((E2, DP), jnp.float32),
        mesh=mesh,
        scratch_types=[
            pltpu.VMEM((RING, GC), jnp.int32),
            pltpu.VMEM((RING, GC), jnp.int32),
            pltpu.VMEM((RING, GC, DP), jnp.float32),
            pltpu.SemaphoreType.DMA,
            pltpu.SemaphoreType.DMA,
            pltpu.SemaphoreType.DMA,
            pltpu.SemaphoreType.DMA,
            pltpu.SemaphoreType.DMA,
        ],
    )
    return f(t1, t2, src, dst)


EB = 4000


def _edge_body(ab_ref, ea_ref, a3_ref, m_ref):
    logits = (
        ab_ref[...]
        + jnp.dot(ea_ref[...], a3_ref[...], preferred_element_type=jnp.float32)
    )
    f = jax.nn.sigmoid(logits[:, :D])
    s = jax.nn.softplus(logits[:, D:])
    m_ref[...] = f * s


def _edge_tc(ab, ea, a3, half):
    off = half * (E2 // EB)
    return pl.pallas_call(
        _edge_body,
        grid=(E2 // EB,),
        in_specs=[
            pl.BlockSpec((EB, DP), lambda i: (i, 0)),
            pl.BlockSpec((EB, DE), lambda i: (i + off, 0)),
            pl.BlockSpec((DE, DP), lambda i: (0, 0)),
        ],
        out_specs=pl.BlockSpec((EB, D), lambda i: (i, 0)),
        out_shape=jax.ShapeDtypeStruct((E2, D), jnp.float32),
    )(ab, ea, a3)


NPC = N // NC
ACC_ROWS = 25088
ZPT = ACC_ROWS // NS
ZC = 56
S_FULL = (NCH // (NS * RING)) * RING
S_TAIL = NCH - S_FULL * NS
OC = 200
NOC = NPC // OC


def _scatter_body(chunk0, m_hbm, src_hbm, msg_hbm, acc,
                  srcbuf, mbuf, idxbuf, zbuf, sem_s, sem_m, sem_sc):
    c = lax.axis_index("c")
    s = lax.axis_index("s")
    nodebase = c * NPC
    sbase = s * S_FULL

    def zrow(r, _):
        for j in range(D // 16):
            zbuf[r, pl.ds(j * 16, 16)] = jnp.zeros((16,), jnp.float32)
        return ()
    lax.fori_loop(0, ZC, zrow, ())
    for j in range(ZPT // ZC):
        pltpu.sync_copy(zbuf, acc.at[pl.ds(s * ZPT + j * ZC, ZC)])
    plsc.subcore_barrier()

    def loads(k, b):
        goff = (chunk0 + sbase + k) * GC
        loff = (sbase + k) * GC
        pltpu.async_copy(src_hbm.at[pl.ds(goff, GC)], srcbuf.at[b], sem_s)
        pltpu.async_copy(m_hbm.at[pl.ds(loff, GC)], mbuf.at[b], sem_m)

    def remap(b):
        sb = srcbuf.at[b]
        ib = idxbuf.at[b]
        for j in range(GC // 16):
            v = sb[pl.ds(j * 16, 16)] - nodebase
            ok = (v >= 0) & (v < NPC)
            ib[pl.ds(j * 16, 16)] = jnp.where(ok, v, NPC)

    for b in range(RING):
        loads(b, b)

    def group(g, _):
        cps = []
        for b in range(RING):
            pltpu.make_async_copy(src_hbm.at[pl.ds(0, GC)], srcbuf.at[b],
                                  sem_s).wait()
            remap(b)
            pltpu.make_async_copy(m_hbm.at[pl.ds(0, GC)], mbuf.at[b],
                                  sem_m).wait()
            cps.append(pltpu.async_copy(mbuf.at[b], acc.at[idxbuf.at[b]],
                                        sem_sc, add=True))
        for b in range(RING):
            k = g * RING + b
            cps[b].wait()

            @pl.when(k + RING < S_FULL)
            def _():
                loads(k + RING, b)
        return ()

    lax.fori_loop(0, S_FULL // RING, group, ())

    @pl.when(s < S_TAIL)
    def _():
        lk = S_FULL * NS + s
        goff = (chunk0 + lk) * GC
        loff = lk * GC
        pltpu.sync_copy(src_hbm.at[pl.ds(goff, GC)], srcbuf.at[0])
        pltpu.sync_copy(m_hbm.at[pl.ds(loff, GC)], mbuf.at[0])
        remap(0)
        pltpu.sync_copy(mbuf.at[0], acc.at[idxbuf.at[0]], add=True)

    plsc.subcore_barrier()

    for i in range(8):
        cid = s * 8 + i

        @pl.when(cid < NOC)
        def _():
            pltpu.sync_copy(acc.at[pl.ds(cid * OC, OC)],
                            msg_hbm.at[pl.ds(nodebase + cid * OC, OC)])


def _scatter_sc(m, src, half):
    mesh = plsc.VectorSubcoreMesh(core_axis_name="c", subcore_axis_name="s")
    f = pl.kernel(
        functools.partial(_scatter_body, half * NCH),
        out_type=jax.ShapeDtypeStruct((N, D), jnp.float32),
        mesh=mesh,
        compiler_params=pltpu.CompilerParams(use_tc_tiling_on_sc=False),
        scratch_types=[
            pltpu.VMEM_SHARED((ACC_ROWS, D), jnp.float32),
            pltpu.VMEM((RING, GC), jnp.int32),
            pltpu.VMEM((RING, GC, D), jnp.float32),
            pltpu.VMEM((RING, GC), jnp.int32),
            pltpu.VMEM((ZC, D), jnp.float32),
            pltpu.SemaphoreType.DMA,
            pltpu.SemaphoreType.DMA,
            pltpu.SemaphoreType.DMA,
        ],
    )
    return f(m, src)


def _stats_body(ma_ref, mb_ref, out_ref):
    @pl.when(pl.program_id(0) == 0)
    def _():
        out_ref[...] = jnp.zeros_like(out_ref)

    blk = ma_ref[...] + mb_ref[...]
    s1 = jnp.sum(blk, axis=0, keepdims=True)
    s2 = jnp.sum(blk * blk, axis=0, keepdims=True)
    out_ref[...] += jnp.concatenate([s1, s2], axis=0)


def _stats_tc(msga, msgb):
    return pl.pallas_call(
        _stats_body,
        grid=(N // NB,),
        in_specs=[pl.BlockSpec((NB, D), lambda i: (i, 0)),
                  pl.BlockSpec((NB, D), lambda i: (i, 0))],
        out_specs=pl.BlockSpec((2, D), lambda i: (0, 0)),
        out_shape=jax.ShapeDtypeStruct((2, D), jnp.float32),
    )(msga, msgb)


def _final_body(x_ref, ma_ref, mb_ref, sums_ref, g_ref, bt_ref, out_ref):
    mean = sums_ref[0:1, :] * (1.0 / N)
    ex2 = sums_ref[1:2, :] * (1.0 / N)
    var = ex2 - mean * mean
    inv = lax.rsqrt(var + 1e-5)
    msg = ma_ref[...] + mb_ref[...]
    normed = (msg - mean) * (inv * g_ref[...]) + bt_ref[...]
    out_ref[...] = jax.nn.softplus(x_ref[...] + normed)


def _final_tc(x, msga, msgb, sums, g, bt):
    return pl.pallas_call(
        _final_body,
        grid=(N // NB,),
        in_specs=[
            pl.BlockSpec((NB, D), lambda i: (i, 0)),
            pl.BlockSpec((NB, D), lambda i: (i, 0)),
            pl.BlockSpec((NB, D), lambda i: (i, 0)),
            pl.BlockSpec((2, D), lambda i: (0, 0)),
            pl.BlockSpec((1, D), lambda i: (0, 0)),
            pl.BlockSpec((1, D), lambda i: (0, 0)),
        ],
        out_specs=pl.BlockSpec((NB, D), lambda i: (i, 0)),
        out_shape=jax.ShapeDtypeStruct((N, D), jnp.float32),
    )(x, msga, msgb, sums, g, bt)


def kernel(x, edge_source, edge_target, edge_attr, Wf, bf, Ws, bs, gamma, beta):
    src = edge_source.astype(jnp.int32)
    dst = edge_target.astype(jnp.int32)
    w1 = jnp.concatenate([Wf[:, :D].T, Ws[:, :D].T], axis=1)
    w2 = jnp.concatenate([Wf[:, D:2 * D].T, Ws[:, D:2 * D].T], axis=1)
    a3 = jnp.concatenate([Wf[:, 2 * D:].T, Ws[:, 2 * D:].T], axis=1)
    b = jnp.concatenate([bf, bs]).reshape(1, DP)

    t1, t2 = _proj_tc(x, w1, w2, b)
    ab0 = _gather_sc(t1, t2, src, dst, 0)
    ab1 = _gather_sc(t1, t2, src, dst, 1)
    m0 = _edge_tc(ab0, edge_attr, a3, 0)
    m1 = _edge_tc(ab1, edge_attr, a3, 1)
    msg0 = _scatter_sc(m0, src, 0)
    msg1 = _scatter_sc(m1, src, 1)
    sums = _stats_tc(msg0, msg1)
    return _final_tc(x, msg0, msg1, sums, gamma.reshape(1, D), beta.reshape(1, D))


# speedup vs baseline: 4.0515x; 1.1030x over previous
import functools

import jax
import jax.numpy as jnp
from jax import lax
from jax.experimental import pallas as pl
from jax.experimental.pallas import tpu as pltpu
from jax.experimental.pallas import tpu_sc as plsc

N = 50000
E = 800000
D = 64
DE = 16
DP = 128

NC = 2
NS = 16
NW = NC * NS

NHALF = 2
E2 = E // NHALF


NB = 1000


def _proj_body(x_ref, w1_ref, w2_ref, b_ref, t1_ref, t2_ref):
    xb = x_ref[...]
    t1_ref[...] = jnp.dot(xb, w1_ref[...], preferred_element_type=jnp.float32)
    t2_ref[...] = (
        jnp.dot(xb, w2_ref[...], preferred_element_type=jnp.float32) + b_ref[...]
    )


def _proj_tc(x, w1, w2, b):
    return pl.pallas_call(
        _proj_body,
        grid=(N // NB,),
        in_specs=[
            pl.BlockSpec((NB, D), lambda i: (i, 0)),
            pl.BlockSpec((D, DP), lambda i: (0, 0)),
            pl.BlockSpec((D, DP), lambda i: (0, 0)),
            pl.BlockSpec((1, DP), lambda i: (0, 0)),
        ],
        out_specs=[
            pl.BlockSpec((NB, DP), lambda i: (i, 0)),
            pl.BlockSpec((NB, DP), lambda i: (i, 0)),
        ],
        out_shape=[
            jax.ShapeDtypeStruct((N, DP), jnp.float32),
            jax.ShapeDtypeStruct((N, DP), jnp.float32),
        ],
    )(x, w1, w2, b)


GC = 128
RING = 3
NCH = E2 // GC
G_FULL = (NCH // (NW * RING)) * RING
G_TAIL = NCH - G_FULL * NW
G_TR = -(-G_TAIL // NW)


def _gather_body(chunk0, t1_hbm, t2_hbm, src_hbm, dst_hbm, ab_hbm,
                 idx_s, idx_d, rows,
                 sem_is, sem_id, sem_g1, sem_g2, sem_wb):
    c = lax.axis_index("c")
    s = lax.axis_index("s")
    w = c * NS + s
    wbase = w * G_FULL

    def idx_load(k, b):
        off = (chunk0 + wbase + k) * GC
        pltpu.async_copy(src_hbm.at[pl.ds(off, GC)], idx_s.at[b], sem_is)
        pltpu.async_copy(dst_hbm.at[pl.ds(off, GC)], idx_d.at[b], sem_id)

    for b in range(RING):
        idx_load(b, b)

    def group(g, _):
        for b in range(RING):
            pltpu.make_async_copy(src_hbm.at[pl.ds(0, GC)], idx_s.at[b],
                                  sem_is).wait()

            @pl.when(g != 0)
            def _():
                pltpu.make_async_copy(rows.at[b],
                                      ab_hbm.at[pl.ds(0, GC)], sem_wb).wait()
            pltpu.async_copy(t1_hbm.at[idx_s.at[b]], rows.at[b], sem_g1)
        for b in range(RING):
            pltpu.make_async_copy(t1_hbm.at[idx_s.at[b]], rows.at[b],
                                  sem_g1).wait()
            pltpu.make_async_copy(dst_hbm.at[pl.ds(0, GC)], idx_d.at[b],
                                  sem_id).wait()
            pltpu.async_copy(t2_hbm.at[idx_d.at[b]], rows.at[b], sem_g2,
                             add=True)
        for b in range(RING):
            k = g * RING + b
            off = (wbase + k) * GC
            pltpu.make_async_copy(t2_hbm.at[idx_d.at[b]], rows.at[b],
                                  sem_g2).wait()
            pltpu.async_copy(rows.at[b], ab_hbm.at[pl.ds(off, GC)], sem_wb)

            @pl.when(k + RING < G_FULL)
            def _():
                idx_load(k + RING, b)
        return ()

    lax.fori_loop(0, G_FULL // RING, group, ())
    for b in range(RING):
        pltpu.make_async_copy(rows.at[b], ab_hbm.at[pl.ds(0, GC)],
                              sem_wb).wait()

    for t in range(G_TR):
        tid = t * NW + w

        @pl.when(tid < G_TAIL)
        def _():
            lk = G_FULL * NW + tid
            off = (chunk0 + lk) * GC
            loff = lk * GC
            pltpu.sync_copy(src_hbm.at[pl.ds(off, GC)], idx_s.at[0])
            pltpu.sync_copy(dst_hbm.at[pl.ds(off, GC)], idx_d.at[0])
            pltpu.async_copy(t1_hbm.at[idx_s.at[0]], rows.at[0], sem_g1).wait()
            pltpu.async_copy(t2_hbm.at[idx_d.at[0]], rows.at[0], sem_g2,
                             add=True).wait()
            pltpu.sync_copy(rows.at[0], ab_hbm.at[pl.ds(loff, GC)])


def _gather_sc(t1, t2, src, dst, half):
    mesh = plsc.VectorSubcoreMesh(core_axis_name="c", subcore_axis_name="s")
    f = pl.kernel(
        functools.partial(_gather_body, half * NCH),
        out_type=jax.ShapeDtypeStruct((E2, DP), jnp.float32),
        mesh=mesh,
        scratch_types=[
            pltpu.VMEM((RING, GC), jnp.int32),
            pltpu.VMEM((RING, GC), jnp.int32),
            pltpu.VMEM((RING, GC, DP), jnp.float32),
            pltpu.SemaphoreType.DMA,
            pltpu.SemaphoreType.DMA,
            pltpu.SemaphoreType.DMA,
            pltpu.SemaphoreType.DMA,
            pltpu.SemaphoreType.DMA,
        ],
    )
    return f(t1, t2, src, dst)


EB = 4000


def _edge_body(ab_ref, ea_ref, a3_ref, m_ref):
    logits = (
        ab_ref[...]
        + jnp.dot(ea_ref[...], a3_ref[...], preferred_element_type=jnp.float32)
    )
    f = jax.nn.sigmoid(logits[:, :D])
    s = jax.nn.softplus(logits[:, D:])
    m_ref[...] = f * s


def _edge_tc(ab, ea, a3, half):
    off = half * (E2 // EB)
    return pl.pallas_call(
        _edge_body,
        grid=(E2 // EB,),
        in_specs=[
            pl.BlockSpec((EB, DP), lambda i: (i, 0)),
            pl.BlockSpec((EB, DE), lambda i: (i + off, 0)),
            pl.BlockSpec((DE, DP), lambda i: (0, 0)),
        ],
        out_specs=pl.BlockSpec((EB, D), lambda i: (i, 0)),
        out_shape=jax.ShapeDtypeStruct((E2, D), jnp.float32),
    )(ab, ea, a3)


DH = D // NC
ACC_ROWS = 50176
ZPT = ACC_ROWS // NS
ZC = 56
S_FULL = (NCH // (NS * RING)) * RING
S_TAIL = NCH - S_FULL * NS
OC = 200
NOC = N // OC


def _scatter_body(chunk0, m_hbm, src_hbm, msg_hbm, acc,
                  srcbuf, mbuf, zbuf, sem_s, sem_m, sem_sc):
    c = lax.axis_index("c")
    s = lax.axis_index("s")
    colbase = c * DH
    sbase = s * S_FULL

    def zrow(r, _):
        for j in range(DH // 16):
            zbuf[r, pl.ds(j * 16, 16)] = jnp.zeros((16,), jnp.float32)
        return ()
    lax.fori_loop(0, ZC, zrow, ())
    for j in range(ZPT // ZC):
        pltpu.sync_copy(zbuf, acc.at[pl.ds(s * ZPT + j * ZC, ZC)])
    plsc.subcore_barrier()

    def loads(k, b):
        goff = (chunk0 + sbase + k) * GC
        loff = (sbase + k) * GC
        pltpu.async_copy(src_hbm.at[pl.ds(goff, GC)], srcbuf.at[b], sem_s)
        pltpu.async_copy(m_hbm.at[pl.ds(loff, GC), pl.ds(colbase, DH)],
                         mbuf.at[b], sem_m)

    for b in range(RING):
        loads(b, b)

    def group(g, _):
        cps = []
        for b in range(RING):
            pltpu.make_async_copy(src_hbm.at[pl.ds(0, GC)], srcbuf.at[b],
                                  sem_s).wait()
            pltpu.make_async_copy(m_hbm.at[pl.ds(0, GC), pl.ds(0, DH)],
                                  mbuf.at[b], sem_m).wait()
            cps.append(pltpu.async_copy(mbuf.at[b], acc.at[srcbuf.at[b]],
                                        sem_sc, add=True))
        for b in range(RING):
            k = g * RING + b
            cps[b].wait()

            @pl.when(k + RING < S_FULL)
            def _():
                loads(k + RING, b)
        return ()

    lax.fori_loop(0, S_FULL // RING, group, ())

    @pl.when(s < S_TAIL)
    def _():
        lk = S_FULL * NS + s
        goff = (chunk0 + lk) * GC
        loff = lk * GC
        pltpu.sync_copy(src_hbm.at[pl.ds(goff, GC)], srcbuf.at[0])
        pltpu.sync_copy(m_hbm.at[pl.ds(loff, GC), pl.ds(colbase, DH)],
                        mbuf.at[0])
        pltpu.sync_copy(mbuf.at[0], acc.at[srcbuf.at[0]], add=True)

    plsc.subcore_barrier()

    for i in range(NOC // NS + 1):
        cid = s * (NOC // NS + 1) + i

        @pl.when(cid < NOC)
        def _():
            pltpu.sync_copy(acc.at[pl.ds(cid * OC, OC)],
                            msg_hbm.at[pl.ds(cid * OC, OC),
                                       pl.ds(colbase, DH)])


def _scatter_sc(m, src, half):
    mesh = plsc.VectorSubcoreMesh(core_axis_name="c", subcore_axis_name="s")
    f = pl.kernel(
        functools.partial(_scatter_body, half * NCH),
        out_type=jax.ShapeDtypeStruct((N, D), jnp.float32),
        mesh=mesh,
        compiler_params=pltpu.CompilerParams(use_tc_tiling_on_sc=False),
        scratch_types=[
            pltpu.VMEM_SHARED((ACC_ROWS, DH), jnp.float32),
            pltpu.VMEM((RING, GC), jnp.int32),
            pltpu.VMEM((RING, GC, DH), jnp.float32),
            pltpu.VMEM((ZC, DH), jnp.float32),
            pltpu.SemaphoreType.DMA,
            pltpu.SemaphoreType.DMA,
            pltpu.SemaphoreType.DMA,
        ],
    )
    return f(m, src)


def _stats_body(ma_ref, mb_ref, out_ref):
    @pl.when(pl.program_id(0) == 0)
    def _():
        out_ref[...] = jnp.zeros_like(out_ref)

    blk = ma_ref[...] + mb_ref[...]
    s1 = jnp.sum(blk, axis=0, keepdims=True)
    s2 = jnp.sum(blk * blk, axis=0, keepdims=True)
    out_ref[...] += jnp.concatenate([s1, s2], axis=0)


def _stats_tc(msga, msgb):
    return pl.pallas_call(
        _stats_body,
        grid=(N // NB,),
        in_specs=[pl.BlockSpec((NB, D), lambda i: (i, 0)),
                  pl.BlockSpec((NB, D), lambda i: (i, 0))],
        out_specs=pl.BlockSpec((2, D), lambda i: (0, 0)),
        out_shape=jax.ShapeDtypeStruct((2, D), jnp.float32),
    )(msga, msgb)


def _final_body(x_ref, ma_ref, mb_ref, sums_ref, g_ref, bt_ref, out_ref):
    mean = sums_ref[0:1, :] * (1.0 / N)
    ex2 = sums_ref[1:2, :] * (1.0 / N)
    var = ex2 - mean * mean
    inv = lax.rsqrt(var + 1e-5)
    msg = ma_ref[...] + mb_ref[...]
    normed = (msg - mean) * (inv * g_ref[...]) + bt_ref[...]
    out_ref[...] = jax.nn.softplus(x_ref[...] + normed)


def _final_tc(x, msga, msgb, sums, g, bt):
    return pl.pallas_call(
        _final_body,
        grid=(N // NB,),
        in_specs=[
            pl.BlockSpec((NB, D), lambda i: (i, 0)),
            pl.BlockSpec((NB, D), lambda i: (i, 0)),
            pl.BlockSpec((NB, D), lambda i: (i, 0)),
            pl.BlockSpec((2, D), lambda i: (0, 0)),
            pl.BlockSpec((1, D), lambda i: (0, 0)),
            pl.BlockSpec((1, D), lambda i: (0, 0)),
        ],
        out_specs=pl.BlockSpec((NB, D), lambda i: (i, 0)),
        out_shape=jax.ShapeDtypeStruct((N, D), jnp.float32),
    )(x, msga, msgb, sums, g, bt)


def kernel(x, edge_source, edge_target, edge_attr, Wf, bf, Ws, bs, gamma, beta):
    src = edge_source.astype(jnp.int32)
    dst = edge_target.astype(jnp.int32)
    w1 = jnp.concatenate([Wf[:, :D].T, Ws[:, :D].T], axis=1)
    w2 = jnp.concatenate([Wf[:, D:2 * D].T, Ws[:, D:2 * D].T], axis=1)
    a3 = jnp.concatenate([Wf[:, 2 * D:].T, Ws[:, 2 * D:].T], axis=1)
    b = jnp.concatenate([bf, bs]).reshape(1, DP)

    t1, t2 = _proj_tc(x, w1, w2, b)
    ab0 = _gather_sc(t1, t2, src, dst, 0)
    ab1 = _gather_sc(t1, t2, src, dst, 1)
    m0 = _edge_tc(ab0, edge_attr, a3, 0)
    m1 = _edge_tc(ab1, edge_attr, a3, 1)
    msg0 = _scatter_sc(m0, src, 0)
    msg1 = _scatter_sc(m1, src, 1)
    sums = _stats_tc(msg0, msg1)
    return _final_tc(x, msg0, msg1, sums, gamma.reshape(1, D), beta.reshape(1, D))
